# baseline (speedup 1.0000x reference)
import functools

import jax
import jax.numpy as jnp
from jax import lax
from jax.experimental import pallas as pl
from jax.experimental.pallas import tpu as pltpu
from jax.experimental.pallas import tpu_sc as plsc

_B, _D, _R = 16384, 128, 1000
_NC, _NS, _L = 2, 16, 16
_NW = _NC * _NS
_RPW = _B // _NW
_C = 128
_NCHUNK = _RPW // _C


def _build():
    mesh = plsc.VectorSubcoreMesh(core_axis_name="c", subcore_axis_name="s")

    @functools.partial(
        pl.kernel,
        mesh=mesh,
        out_type=jax.ShapeDtypeStruct((_B,), jnp.float32),
        compiler_params=pltpu.CompilerParams(needs_layout_passes=False),
        scratch_types=[
            pltpu.VMEM((_C,), jnp.int32),
            pltpu.VMEM((_C, _D), jnp.float32),
            pltpu.VMEM((_C, _D), jnp.float32),
            pltpu.VMEM((_C, _D), jnp.float32),
            pltpu.VMEM((_C,), jnp.float32),
            pltpu.VMEM((_L,), jnp.float32),
            pltpu.SemaphoreType.DMA,
        ],
    )
    def k(s_hbm, r_hbm, o_hbm, rel_hbm, scale_hbm, out_hbm,
          idx_v, w_v, s_v, o_v, out_v, scale_v, sem):
        wid = lax.axis_index("s") * _NC + lax.axis_index("c")
        base = wid * _RPW
        pltpu.sync_copy(scale_hbm, scale_v)
        scale_vec = scale_v[...]
        lane = lax.iota(jnp.int32, _L)

        def do_chunk(c, carry):
            cb = base + c * _C
            pltpu.sync_copy(r_hbm.at[pl.ds(cb, _C)], idx_v)
            gather = pltpu.async_copy(rel_hbm.at[idx_v], w_v, sem)
            pltpu.sync_copy(s_hbm.at[pl.ds(cb, _C), :], s_v)
            pltpu.sync_copy(o_hbm.at[pl.ds(cb, _C), :], o_v)
            gather.wait()

            def do_group(g, carry2):
                res = jnp.zeros((_L,), jnp.float32)
                for kk in range(_L):
                    i = g * _L + kk
                    acc = (s_v[i, pl.ds(0, _L)]
                           * w_v[i, pl.ds(0, _L)]
                           * o_v[i, pl.ds(0, _L)])
                    for j in range(1, _D // _L):
                        acc = acc + (s_v[i, pl.ds(j * _L, _L)]
                                     * w_v[i, pl.ds(j * _L, _L)]
                                     * o_v[i, pl.ds(j * _L, _L)])
                    acc = acc * scale_vec
                    rs = jnp.sum(acc)
                    res = jnp.where(lane == kk, rs, res)
                out_v[pl.ds(g * _L, _L)] = res
                return carry2

            lax.fori_loop(0, _C // _L, do_group, 0)
            pltpu.sync_copy(out_v, out_hbm.at[pl.ds(cb, _C)])
            return carry

        lax.fori_loop(0, _NCHUNK, do_chunk, 0)

    return k


_sc_kernel = _build()


def kernel(s, r, o, rel, scale):
    r32 = r.astype(jnp.int32)
    scale_vec = jnp.full((_L,), scale, dtype=jnp.float32)
    return _sc_kernel(s, r32, o, rel, scale_vec)


# speedup vs baseline: 1.7180x; 1.7180x over previous
import functools

import jax
import jax.numpy as jnp
from jax import lax
from jax.experimental import pallas as pl
from jax.experimental.pallas import tpu as pltpu
from jax.experimental.pallas import tpu_sc as plsc

_B, _D, _R = 16384, 128, 1000
_NC, _NS, _L = 2, 16, 16
_NW = _NC * _NS
_RPW = _B // _NW
_C = 128
_NCHUNK = _RPW // _C


def _build():
    mesh = plsc.VectorSubcoreMesh(core_axis_name="c", subcore_axis_name="s")

    @functools.partial(
        pl.kernel,
        mesh=mesh,
        out_type=jax.ShapeDtypeStruct((_B,), jnp.float32),
        compiler_params=pltpu.CompilerParams(needs_layout_passes=False),
        scratch_types=[
            pltpu.VMEM((_NCHUNK, _C), jnp.int32),
            pltpu.VMEM((2, _C, _D), jnp.float32),
            pltpu.VMEM((2, _C, _D), jnp.float32),
            pltpu.VMEM((2, _C, _D), jnp.float32),
            pltpu.VMEM((_C + _L,), jnp.float32),
            pltpu.VMEM((_L,), jnp.float32),
            pltpu.SemaphoreType.DMA,
            pltpu.SemaphoreType.DMA,
        ],
    )
    def k(s_hbm, r_hbm, o_hbm, rel_hbm, scale_hbm, out_hbm,
          idx_v, w_v, s_v, o_v, out_v, scale_v, sem0, sem1):
        wid = lax.axis_index("s") * _NC + lax.axis_index("c")
        base = wid * _RPW
        pltpu.sync_copy(scale_hbm, scale_v)
        pltpu.sync_copy(r_hbm.at[pl.ds(wid * _NCHUNK, _NCHUNK), :], idx_v)
        scale_vec = scale_v[...]
        lane = lax.iota(jnp.int32, _L)
        last_lane = lane == (_L - 1)
        sems = (sem0, sem1)

        def issue(c):
            buf = c % 2
            cb = base + c * _C
            sem = sems[buf]
            return (
                pltpu.async_copy(rel_hbm.at[idx_v.at[c]], w_v.at[buf], sem),
                pltpu.async_copy(s_hbm.at[pl.ds(cb, _C), :], s_v.at[buf], sem),
                pltpu.async_copy(o_hbm.at[pl.ds(cb, _C), :], o_v.at[buf], sem),
            )

        pending = issue(0)
        for c in range(_NCHUNK):
            buf = c % 2
            cb = base + c * _C
            for d in pending:
                d.wait()
            if c + 1 < _NCHUNK:
                pending = issue(c + 1)

            wb, sb, ob = w_v.at[buf], s_v.at[buf], o_v.at[buf]

            def row(i, carry, wb=wb, sb=sb, ob=ob):
                acc = (sb[i, pl.ds(0, _L)]
                       * wb[i, pl.ds(0, _L)]
                       * ob[i, pl.ds(0, _L)])
                for j in range(1, _D // _L):
                    acc = acc + (sb[i, pl.ds(j * _L, _L)]
                                 * wb[i, pl.ds(j * _L, _L)]
                                 * ob[i, pl.ds(j * _L, _L)])
                cum = plsc.cumsum(acc)
                plsc.store_compressed(out_v.at[pl.ds(i, _L)], cum,
                                      mask=last_lane)
                return carry

            lax.fori_loop(0, _C, row, 0, unroll=4)

            for jj in range(_C // _L):
                sl = pl.ds(jj * _L, _L)
                out_v[sl] = out_v[sl] * scale_vec
            pltpu.sync_copy(out_v.at[pl.ds(0, _C)], out_hbm.at[pl.ds(cb, _C)])

    return k


_sc_kernel = _build()


def kernel(s, r, o, rel, scale):
    r32 = r.astype(jnp.int32).reshape(_B // _C, _C)
    scale_vec = jnp.full((_L,), scale, dtype=jnp.float32)
    return _sc_kernel(s, r32, o, rel, scale_vec)


# speedup vs baseline: 1.8433x; 1.0729x over previous
import functools

import jax
import jax.numpy as jnp
from jax import lax
from jax.experimental import pallas as pl
from jax.experimental.pallas import tpu as pltpu
from jax.experimental.pallas import tpu_sc as plsc

_B, _D, _R = 16384, 128, 1000
_NC, _NS, _L = 2, 16, 16
_NW = _NC * _NS
_RPW = _B // _NW
_C = 128
_NCHUNK = _RPW // _C


def _build():
    mesh = plsc.VectorSubcoreMesh(core_axis_name="c", subcore_axis_name="s")

    @functools.partial(
        pl.kernel,
        mesh=mesh,
        out_type=jax.ShapeDtypeStruct((_B,), jnp.float32),
        compiler_params=pltpu.CompilerParams(needs_layout_passes=False),
        scratch_types=[
            pltpu.VMEM((_NCHUNK, _C), jnp.int32),
            pltpu.VMEM((2, _C, _D), jnp.float32),
            pltpu.VMEM((2, _C, _D), jnp.float32),
            pltpu.VMEM((2, _C, _D), jnp.float32),
            pltpu.VMEM((_C + _L,), jnp.float32),
            pltpu.VMEM((_L,), jnp.float32),
            pltpu.SemaphoreType.DMA,
            pltpu.SemaphoreType.DMA,
        ],
    )
    def k(s_hbm, r_hbm, o_hbm, rel_hbm, scale_hbm, out_hbm,
          idx_v, w_v, s_v, o_v, out_v, scale_v, sem0, sem1):
        wid = lax.axis_index("s") * _NC + lax.axis_index("c")
        base = wid * _RPW
        pltpu.sync_copy(scale_hbm, scale_v)
        pltpu.sync_copy(r_hbm.at[pl.ds(wid * _NCHUNK, _NCHUNK), :], idx_v)
        scale_vec = scale_v[...]
        lane = lax.iota(jnp.int32, _L)
        last_lane = lane == (_L - 1)
        sems = (sem0, sem1)

        def issue(c):
            buf = c % 2
            cb = base + c * _C
            sem = sems[buf]
            return (
                pltpu.async_copy(rel_hbm.at[idx_v.at[c]], w_v.at[buf], sem),
                pltpu.async_copy(s_hbm.at[pl.ds(cb, _C), :], s_v.at[buf], sem),
                pltpu.async_copy(o_hbm.at[pl.ds(cb, _C), :], o_v.at[buf], sem),
            )

        pending = issue(0)
        for c in range(_NCHUNK):
            buf = c % 2
            cb = base + c * _C
            for d in pending:
                d.wait()
            if c + 1 < _NCHUNK:
                pending = issue(c + 1)

            wb, sb, ob = w_v.at[buf], s_v.at[buf], o_v.at[buf]

            @plsc.parallel_loop(0, _C, 1, unroll=4)
            def row(i, wb=wb, sb=sb, ob=ob):
                acc = (sb[i, pl.ds(0, _L)]
                       * wb[i, pl.ds(0, _L)]
                       * ob[i, pl.ds(0, _L)])
                for j in range(1, _D // _L):
                    acc = acc + (sb[i, pl.ds(j * _L, _L)]
                                 * wb[i, pl.ds(j * _L, _L)]
                                 * ob[i, pl.ds(j * _L, _L)])
                cum = plsc.cumsum(acc)
                plsc.store_compressed(out_v.at[pl.ds(i, _L)], cum,
                                      mask=last_lane)

            for jj in range(_C // _L):
                sl = pl.ds(jj * _L, _L)
                out_v[sl] = out_v[sl] * scale_vec
            pltpu.sync_copy(out_v.at[pl.ds(0, _C)], out_hbm.at[pl.ds(cb, _C)])

    return k


_sc_kernel = _build()


def kernel(s, r, o, rel, scale):
    r32 = r.astype(jnp.int32).reshape(_B // _C, _C)
    scale_vec = jnp.full((_L,), scale, dtype=jnp.float32)
    return _sc_kernel(s, r32, o, rel, scale_vec)
